# baseline (speedup 1.0000x reference)
import functools

import jax
import jax.numpy as jnp
import numpy as np
from jax import lax
from jax.experimental import pallas as pl
from jax.experimental.pallas import tpu as pltpu
from jax.experimental.pallas import tpu_sc as plsc

D_MODEL = 768
MAX_LEN = 8192

NC = 2
NS = 16
L = 16
NW = NC * NS


@functools.lru_cache(maxsize=4)
def _make_pe(seq_len, d_model):
    pos = np.arange(seq_len, dtype=np.float32)[:, None]
    i = np.arange(0, d_model, 2, dtype=np.float32)
    div = np.power(np.float32(10000.0), i / np.float32(d_model))
    pe = np.zeros((seq_len, d_model), dtype=np.float32)
    pe[:, 0::2] = np.sin(pos / div)
    pe[:, 1::2] = np.cos(pos / div)
    return jnp.asarray(pe)


@functools.partial(jax.jit, static_argnames=("bsz", "seq", "dim", "ch"))
def _embed_sc(x, tok_table, pe, *, bsz, seq, dim, ch):
    ppw = seq // NW
    nch = ppw // ch

    nv = dim // L
    steps = nch * bsz

    mesh = plsc.VectorSubcoreMesh(
        core_axis_name="c", subcore_axis_name="s",
        num_cores=NC, num_subcores=NS)

    @functools.partial(
        pl.kernel,
        mesh=mesh,
        out_type=jax.ShapeDtypeStruct((bsz, seq, dim), jnp.float32),
        scratch_types=[
            pltpu.VMEM((bsz, ppw), jnp.int32),
            pltpu.VMEM((ch, dim), jnp.float32),
            pltpu.VMEM((ch, dim), jnp.float32),
            pltpu.VMEM((ch, dim), jnp.float32),
            pltpu.VMEM((ch, dim), jnp.float32),
            pltpu.VMEM((ch, dim), jnp.float32),
            pltpu.SemaphoreType.DMA,
            pltpu.SemaphoreType.DMA,
            pltpu.SemaphoreType.DMA,
            pltpu.SemaphoreType.DMA,
            pltpu.SemaphoreType.DMA,
            pltpu.SemaphoreType.DMA,
            pltpu.SemaphoreType.DMA,
            pltpu.SemaphoreType.DMA,
        ],
    )
    def k(x_ref, tab_ref, pe_ref, out_ref, idx_v,
          rows0, rows1, rows2, pe0, pe1,
          gs0, gs1, gs2, os0, os1, os2, ps0, ps1):
        rows, pes = [rows0, rows1, rows2], [pe0, pe1]
        gsem, osem, psem = [gs0, gs1, gs2], [os0, os1, os2], [ps0, ps1]
        wid = lax.axis_index("s") * NC + lax.axis_index("c")
        p0 = wid * ppw

        pltpu.sync_copy(x_ref.at[:, pl.ds(p0, ppw)], idx_v)

        def gather_start(s):
            pc, b = divmod(s, bsz)
            return pltpu.async_copy(
                tab_ref.at[idx_v.at[b, pl.ds(pc * ch, ch)]],
                rows[s % 3], gsem[s % 3])

        def pe_start(pc):
            return pltpu.async_copy(
                pe_ref.at[pl.ds(p0 + pc * ch, ch)], pes[pc % 2], psem[pc % 2])

        gh = {0: gather_start(0)}
        if steps > 1:
            gh[1] = gather_start(1)
        ph = {0: pe_start(0)}
        oh = {}
        for s in range(steps):
            pc, b = divmod(s, bsz)
            gh[s].wait()
            if b == 0:
                ph[pc].wait()
                if pc + 1 < nch:
                    ph[pc + 1] = pe_start(pc + 1)
            if s + 2 < steps:
                if s >= 1:
                    oh[s - 1].wait()
                gh[s + 2] = gather_start(s + 2)
            rv, pv = rows[s % 3], pes[pc % 2]

            @plsc.parallel_loop(0, ch, unroll=2)
            def _add(r, rv=rv, pv=pv):
                for j in range(nv):
                    sl = pl.ds(j * L, L)
                    plsc.addupdate(rv.at[r, sl], pv[r, sl])

            oh[s] = pltpu.async_copy(
                rv, out_ref.at[b, pl.ds(p0 + pc * ch, ch)], osem[s % 3])
        for s in range(max(0, steps - 3), steps):
            oh[s].wait()

    return k(x, tok_table, pe)


def kernel(x, tok_table):
    bsz, seq = x.shape
    seq = min(seq, MAX_LEN)
    x = x[:, :seq]
    dim = tok_table.shape[1]
    pe = _make_pe(seq, dim)
    return _embed_sc(x, tok_table, pe, bsz=bsz, seq=seq, dim=dim, ch=32)


# speedup vs baseline: 1.0259x; 1.0259x over previous
import functools

import jax
import jax.numpy as jnp
import numpy as np
from jax import lax
from jax.experimental import pallas as pl
from jax.experimental.pallas import tpu as pltpu
from jax.experimental.pallas import tpu_sc as plsc

D_MODEL = 768
MAX_LEN = 8192

NC = 2
NS = 16
L = 16
NW = NC * NS


@functools.lru_cache(maxsize=4)
def _make_pe(seq_len, d_model):
    pos = np.arange(seq_len, dtype=np.float32)[:, None]
    i = np.arange(0, d_model, 2, dtype=np.float32)
    div = np.power(np.float32(10000.0), i / np.float32(d_model))
    pe = np.zeros((seq_len, d_model), dtype=np.float32)
    pe[:, 0::2] = np.sin(pos / div)
    pe[:, 1::2] = np.cos(pos / div)
    return jnp.asarray(pe)


@functools.partial(jax.jit, static_argnames=("bsz", "seq", "dim", "ch"))
def _embed_sc(x, tok_table, pe, *, bsz, seq, dim, ch):
    ppw = seq // NW
    nch = ppw // ch

    nv = dim // L
    steps = nch * bsz

    mesh = plsc.VectorSubcoreMesh(
        core_axis_name="c", subcore_axis_name="s",
        num_cores=NC, num_subcores=NS)

    @functools.partial(
        pl.kernel,
        mesh=mesh,
        out_type=jax.ShapeDtypeStruct((bsz, seq, dim), jnp.float32),
        scratch_types=[
            pltpu.VMEM((bsz, ppw), jnp.int32),
            pltpu.VMEM((ch, dim), jnp.float32),
            pltpu.VMEM((ch, dim), jnp.float32),
            pltpu.VMEM((ch, dim), jnp.float32),
            pltpu.VMEM((ch, dim), jnp.float32),
            pltpu.VMEM((ch, dim), jnp.float32),
            pltpu.SemaphoreType.DMA,
            pltpu.SemaphoreType.DMA,
            pltpu.SemaphoreType.DMA,
            pltpu.SemaphoreType.DMA,
            pltpu.SemaphoreType.DMA,
            pltpu.SemaphoreType.DMA,
            pltpu.SemaphoreType.DMA,
            pltpu.SemaphoreType.DMA,
        ],
    )
    def k(x_ref, tab_ref, pe_ref, out_ref, idx_v,
          rows0, rows1, rows2, pe0, pe1,
          gs0, gs1, gs2, os0, os1, os2, ps0, ps1):
        rows, pes = [rows0, rows1, rows2], [pe0, pe1]
        gsem, osem, psem = [gs0, gs1, gs2], [os0, os1, os2], [ps0, ps1]
        wid = lax.axis_index("s") * NC + lax.axis_index("c")
        p0 = wid * ppw

        pltpu.sync_copy(x_ref.at[:, pl.ds(p0, ppw)], idx_v)

        def gather_start(s):
            pc, b = divmod(s, bsz)
            return pltpu.async_copy(
                tab_ref.at[idx_v.at[b, pl.ds(pc * ch, ch)]],
                rows[s % 3], gsem[s % 3])

        def pe_start(pc):
            return pltpu.async_copy(
                pe_ref.at[pl.ds(p0 + pc * ch, ch)], pes[pc % 2], psem[pc % 2])

        gh = {0: gather_start(0)}
        if steps > 1:
            gh[1] = gather_start(1)
        ph = {0: pe_start(0)}
        oh = {}
        for s in range(steps):
            pc, b = divmod(s, bsz)
            gh[s].wait()
            if b == 0:
                ph[pc].wait()
                if pc + 1 < nch:
                    ph[pc + 1] = pe_start(pc + 1)
            if s + 2 < steps:
                if s >= 1:
                    oh[s - 1].wait()
                gh[s + 2] = gather_start(s + 2)
            rv, pv = rows[s % 3], pes[pc % 2]

            @plsc.parallel_loop(0, ch)
            def _add(r, rv=rv, pv=pv):
                for j in range(nv):
                    sl = pl.ds(j * L, L)
                    plsc.addupdate(rv.at[r, sl], pv[r, sl])

            oh[s] = pltpu.async_copy(
                rv, out_ref.at[b, pl.ds(p0 + pc * ch, ch)], osem[s % 3])
        for s in range(max(0, steps - 3), steps):
            oh[s].wait()

    return k(x, tok_table, pe)


def kernel(x, tok_table):
    bsz, seq = x.shape
    seq = min(seq, MAX_LEN)
    x = x[:, :seq]
    dim = tok_table.shape[1]
    pe = _make_pe(seq, dim)
    return _embed_sc(x, tok_table, pe, bsz=bsz, seq=seq, dim=dim, ch=32)
